# Initial kernel scaffold; baseline (speedup 1.0000x reference)
import functools

import jax
import jax.numpy as jnp
from jax import lax
from jax.experimental import pallas as pl
from jax.experimental.pallas import tpu as pltpu
from jax.experimental.pallas import tpu_sc as plsc

N_USERS = 50000
N_ITEMS = 20000
N_ENTITIES = 50000
N_REL = 17
EMB = 64
HOPS = 2
B = 4096
NEG = 16
MARGIN = 0.8
DECAY = 1e-4
ANGLE_W = 0.5
ANGLE_DROP = 0.5

NCORES = 2
NTILES = 16
EP = 802816
SUPER = 1024
CHUNK = 128
NSUPER = EP // (NTILES * SUPER)

HALF_E = 25088
HALF_I = 10112
ACC_E = 26624
ACC_I = 12288
GARB = 63

_f32 = jnp.float32
_i32 = jnp.int32


@functools.lru_cache(maxsize=None)
def _seg_op(n_half, acc_rows, mode, with_deg):
    mesh = plsc.VectorSubcoreMesh(core_axis_name="c", subcore_axis_name="s")
    out_type = [jax.ShapeDtypeStruct((NCORES, n_half, EMB), _f32)]
    if with_deg:
        out_type.append(jax.ShapeDtypeStruct((NCORES, n_half, 8), _f32))
    scratch = [
        pltpu.VMEM((SUPER,), _i32),
        pltpu.VMEM((SUPER,), _f32 if mode == "scalar" else _i32),
        pltpu.VMEM((CHUNK,), _i32),
        pltpu.VMEM((CHUNK, EMB), _f32),
        pltpu.VMEM_SHARED((acc_rows, EMB), _f32),
        pltpu.SemaphoreType.DMA,
    ]
    if mode == "rel":
        scratch.append(pltpu.VMEM((N_REL - 1, EMB), _f32))
    if with_deg:
        scratch.append(pltpu.VMEM((CHUNK, 8), _f32))
        scratch.append(pltpu.VMEM_SHARED((acc_rows, 8), _f32))

    def body(*refs):
        it = iter(refs)
        table = next(it); src = next(it); dst = next(it); w = next(it)
        rel = next(it) if mode == "rel" else None
        ones_h = next(it) if with_deg else None
        z64 = next(it); z8 = next(it)
        out = next(it)
        dout = next(it) if with_deg else None
        sbuf = next(it); wbuf = next(it); dbuf = next(it); rows = next(it)
        acc = next(it); sem = next(it)
        relbuf = next(it) if mode == "rel" else None
        onesbuf = next(it) if with_deg else None
        dacc = next(it) if with_deg else None

        c = lax.axis_index("c")
        s = lax.axis_index("s")

        for j in range(acc_rows // (NTILES * 2048)):
            base = (j * NTILES + s) * 2048
            pltpu.sync_copy(z64, acc.at[pl.ds(base, 2048)])
            if with_deg:
                pltpu.sync_copy(z8, dacc.at[pl.ds(base, 2048)])
        if mode == "rel":
            pltpu.sync_copy(rel, relbuf)
        if with_deg:
            pltpu.sync_copy(ones_h, onesbuf)
        plsc.subcore_barrier()

        half_base = c * n_half

        def chunk_body(k, sj):
            base = sj + k * CHUNK
            pltpu.sync_copy(dst.at[pl.ds(base, CHUNK)], dbuf)
            pltpu.async_copy(
                table.at[sbuf.at[pl.ds(k * CHUNK, CHUNK)]], rows, sem).wait()
            for g in range(CHUNK // 16):
                dv = dbuf[pl.ds(g * 16, 16)]
                loc = dv - half_base
                ok = (loc >= 0) & (loc < n_half)
                garb = n_half + (dv & GARB)
                dbuf[pl.ds(g * 16, 16)] = jnp.where(ok, loc, garb)
            for m in range(CHUNK):
                if mode == "scalar":
                    wm = wbuf[k * CHUNK + m]
                    for q in range(EMB // 16):
                        sl = rows[m, pl.ds(q * 16, 16)]
                        rows[m, pl.ds(q * 16, 16)] = sl * wm
                else:
                    tm = wbuf[k * CHUNK + m] - 1
                    for q in range(EMB // 16):
                        sl = rows[m, pl.ds(q * 16, 16)]
                        rv = relbuf[tm, pl.ds(q * 16, 16)]
                        rows[m, pl.ds(q * 16, 16)] = sl * rv
            pltpu.sync_copy(rows, acc.at[dbuf], add=True)
            if with_deg:
                pltpu.sync_copy(onesbuf, dacc.at[dbuf], add=True)
            return sj

        def super_body(j, carry):
            sbase = (s * NSUPER + j) * SUPER
            pltpu.sync_copy(src.at[pl.ds(sbase, SUPER)], sbuf)
            pltpu.sync_copy(w.at[pl.ds(sbase, SUPER)], wbuf)
            lax.fori_loop(0, SUPER // CHUNK, chunk_body, sbase,
                          unroll=False)
            return carry

        lax.fori_loop(0, NSUPER, super_body, 0, unroll=False)
        plsc.subcore_barrier()

        nchunks = n_half // CHUNK
        for j in range(-(-nchunks // NTILES)):
            cid = j * NTILES + s

            @pl.when(cid < nchunks)
            def _():
                r0 = cid * CHUNK
                pltpu.sync_copy(acc.at[pl.ds(r0, CHUNK)],
                                out.at[c, pl.ds(r0, CHUNK)])
                if with_deg:
                    pltpu.sync_copy(dacc.at[pl.ds(r0, CHUNK)],
                                    dout.at[c, pl.ds(r0, CHUNK)])

    return pl.kernel(body, out_type=out_type, mesh=mesh,
                     scratch_types=scratch)


def _seg(table, src, dst, w, n_half, acc_rows, rel=None, with_deg=False):
    z64 = jnp.zeros((2048, EMB), _f32)
    z8 = jnp.zeros((2048, 8), _f32)
    mode = "rel" if rel is not None else "scalar"
    k = _seg_op(n_half, acc_rows, mode, with_deg)
    args = [table, src, dst, w]
    if rel is not None:
        args.append(rel)
    if with_deg:
        ones = jnp.zeros((CHUNK, 8), _f32).at[:, 0].set(1.0)
        args.append(ones)
    args += [z64, z8]
    out = k(*args)
    return out if isinstance(out, (tuple, list)) else (out,)


@functools.lru_cache(maxsize=None)
def _gather_op(n_rows, n_idx):
    mesh = plsc.VectorSubcoreMesh(core_axis_name="c", subcore_axis_name="s")
    nchunks = n_idx // (NCORES * NTILES * CHUNK)

    def body(table, idx, out, ibuf, rows, sem):
        c = lax.axis_index("c")
        s = lax.axis_index("s")
        wid = s * NCORES + c

        def chunk(j, carry):
            base = (wid * nchunks + j) * CHUNK
            pltpu.sync_copy(idx.at[pl.ds(base, CHUNK)], ibuf)
            pltpu.async_copy(table.at[ibuf], rows, sem).wait()
            pltpu.sync_copy(rows, out.at[pl.ds(base, CHUNK)])
            return carry

        lax.fori_loop(0, nchunks, chunk, 0, unroll=False)

    return pl.kernel(
        body,
        out_type=jax.ShapeDtypeStruct((n_idx, EMB), _f32),
        mesh=mesh,
        scratch_types=[pltpu.VMEM((CHUNK,), _i32),
                       pltpu.VMEM((CHUNK, EMB), _f32),
                       pltpu.SemaphoreType.DMA])


def _gather(table, idx):
    return _gather_op(table.shape[0], idx.shape[0])(table, idx)


def _t1_body(agg_ref, deg_ref, res_ref, e1_ref, out_ref):
    d = jnp.maximum(deg_ref[:, 0:1], 1.0)
    x = agg_ref[...] / d
    e1_ref[...] = x
    nrm = jnp.sqrt(jnp.sum(x * x, axis=1, keepdims=True)) + 1e-8
    out_ref[...] = res_ref[...] + x / nrm


def _t2_body(x_ref, res_ref, out_ref):
    x = x_ref[...]
    nrm = jnp.sqrt(jnp.sum(x * x, axis=1, keepdims=True)) + 1e-8
    out_ref[...] = res_ref[...] + x / nrm


@functools.lru_cache(maxsize=None)
def _t1_call(n_rows):
    blk = 512
    grid = n_rows // blk
    return pl.pallas_call(
        _t1_body,
        grid=(grid,),
        in_specs=[pl.BlockSpec((blk, EMB), lambda i: (i, 0)),
                  pl.BlockSpec((blk, 8), lambda i: (i, 0)),
                  pl.BlockSpec((blk, EMB), lambda i: (i, 0))],
        out_specs=[pl.BlockSpec((blk, EMB), lambda i: (i, 0)),
                   pl.BlockSpec((blk, EMB), lambda i: (i, 0))],
        out_shape=[jax.ShapeDtypeStruct((n_rows, EMB), _f32),
                   jax.ShapeDtypeStruct((n_rows, EMB), _f32)])


@functools.lru_cache(maxsize=None)
def _t2_call(n_rows):
    blk = 512
    grid = n_rows // blk
    return pl.pallas_call(
        _t2_body,
        grid=(grid,),
        in_specs=[pl.BlockSpec((blk, EMB), lambda i: (i, 0)),
                  pl.BlockSpec((blk, EMB), lambda i: (i, 0))],
        out_specs=pl.BlockSpec((blk, EMB), lambda i: (i, 0)),
        out_shape=jax.ShapeDtypeStruct((n_rows, EMB), _f32))


def _loss_body(u_ref, pe_ref, pi_ref, ne_ref, ni_ref, l1_ref, sq_ref):
    i = pl.program_id(0)
    u_e = u_ref[...]
    pe = pe_ref[...]
    pi = pi_ref[...]
    ne = ne_ref[...]
    ni = ni_ref[...]
    u = u_e / (jnp.sqrt(jnp.sum(u_e * u_e, -1, keepdims=True)) + 1e-8)
    ps = pe + pi
    p = ps / (jnp.sqrt(jnp.sum(ps * ps, -1, keepdims=True)) + 1e-8)
    pos_score = jnp.sum(u * p, -1)
    ns = ne + ni
    n = ns / (jnp.sqrt(jnp.sum(ns * ns, -1, keepdims=True)) + 1e-8)
    neg_score = jnp.sum(u[:, None, :] * n, -1)
    hinge = (jnp.maximum(1.0 - pos_score, 0.0)
             + jnp.mean(jnp.maximum(neg_score - MARGIN, 0.0), -1))
    part1 = jnp.sum(hinge)
    sq = (jnp.sum(u_e * u_e) + jnp.sum(pe * pe) + jnp.sum(pi * pi)
          + jnp.sum(ne * ne) + jnp.sum(ni * ni))

    @pl.when(i == 0)
    def _():
        l1_ref[0, 0] = 0.0
        sq_ref[0, 0] = 0.0

    l1_ref[0, 0] += part1
    sq_ref[0, 0] += sq


def _loss_call():
    blk = 512
    grid = B // blk
    return pl.pallas_call(
        _loss_body,
        grid=(grid,),
        in_specs=[pl.BlockSpec((blk, EMB), lambda i: (i, 0)),
                  pl.BlockSpec((blk, EMB), lambda i: (i, 0)),
                  pl.BlockSpec((blk, EMB), lambda i: (i, 0)),
                  pl.BlockSpec((blk, NEG, EMB), lambda i: (i, 0, 0)),
                  pl.BlockSpec((blk, NEG, EMB), lambda i: (i, 0, 0))],
        out_specs=[pl.BlockSpec((1, 1), lambda i: (0, 0)),
                   pl.BlockSpec((1, 1), lambda i: (0, 0))],
        out_shape=[jax.ShapeDtypeStruct((1, 1), _f32),
                   jax.ShapeDtypeStruct((1, 1), _f32)])


def _acos_poly(x):
    a = jnp.abs(x)
    s = jnp.sqrt(1.0 - a)
    p = jnp.float32(-0.0012624911)
    for cc in (0.0066700901, -0.0170881256, 0.0308918810, -0.0501743046,
               0.0889789874, -0.2145988016, 1.5707963050):
        p = p * a + jnp.float32(cc)
    b = s * p
    return jnp.where(x >= 0, b, jnp.float32(jnp.pi) - b)


def _angle_body(t_real, h_ref, t_ref, out_ref):
    i = pl.program_id(0)
    blk = h_ref.shape[0]
    h = h_ref[...] * ANGLE_DROP
    t = t_ref[...] * ANGLE_DROP
    eps = 1e-6
    k_const = 0.1
    hh = jnp.sum(h * h, -1)
    tt = jnp.sum(t * t, -1)
    dot = jnp.sum(h * t, -1)
    d = h - t
    edist = jnp.sqrt(jnp.sum(d * d, -1))
    nu = jnp.sqrt(hh)
    num = dot * (1.0 + hh) - hh * (1.0 + tt)
    den = nu * edist * jnp.sqrt(jnp.clip(1.0 + tt * hh - 2.0 * dot, eps)) + eps
    ang = _acos_poly(jnp.clip(num / den, -1.0 + eps, 1.0 - eps))
    sqnu = jnp.clip(hh, 0.0, 1.0 - eps)
    asin_arg = jnp.clip(k_const * (1.0 - sqnu) / jnp.sqrt(sqnu + eps),
                        -1.0 + eps, 1.0 - eps)
    half = jnp.float32(jnp.pi / 2) - _acos_poly(asin_arg)
    val = jnp.maximum(ang - half, 0.0)
    rid = i * blk + lax.broadcasted_iota(_i32, (blk,), 0)
    part = jnp.sum(jnp.where(rid < t_real, val, 0.0))

    @pl.when(i == 0)
    def _():
        out_ref[0, 0] = 0.0

    out_ref[0, 0] += part


@functools.lru_cache(maxsize=None)
def _angle_call(tp, t_real):
    blk = 2048
    grid = tp // blk
    return pl.pallas_call(
        functools.partial(_angle_body, t_real),
        grid=(grid,),
        in_specs=[pl.BlockSpec((blk, EMB), lambda i: (i, 0)),
                  pl.BlockSpec((blk, EMB), lambda i, g=grid: (g + i, 0))],
        out_specs=pl.BlockSpec((1, 1), lambda i: (0, 0)),
        out_shape=jax.ShapeDtypeStruct((1, 1), _f32))


def _pad_rows(x, n):
    return jnp.zeros((n, EMB), _f32).at[: x.shape[0]].set(x)


def _pad_edges(x, val, dtype):
    return jnp.concatenate(
        [x.astype(dtype), jnp.full((EP - x.shape[0],), val, dtype)])


def kernel(user, pos_item, neg_item, all_embed, item_emb_cf, rel_emb,
           edge_index, edge_type, ui_rows, ui_cols, ui_vals,
           tri_head, tri_tail):
    user_emb = all_embed[:N_USERS]
    entity_emb = all_embed[N_USERS:]

    headp = _pad_edges(edge_index[0], -1, _i32)
    tailp = _pad_edges(edge_index[1], 0, _i32)
    typep = _pad_edges(edge_type, 1, _i32)
    urp = _pad_edges(ui_rows, -1, _i32)
    ucp = _pad_edges(ui_cols, -1, _i32)
    uvp = _pad_edges(ui_vals, 0.0, _f32)

    e_res = _pad_rows(entity_emb, 2 * HALF_E)
    u_res = _pad_rows(user_emb, 2 * HALF_E)
    i_res = _pad_rows(item_emb_cf, 2 * HALF_I)

    etab = entity_emb
    itab = item_emb_cf
    deg = None
    for hop in range(HOPS):
        if hop == 0:
            eagg, deg = _seg(etab, tailp, headp, typep, HALF_E, ACC_E,
                             rel=rel_emb, with_deg=True)
            deg = deg.reshape(2 * HALF_E, 8)
        else:
            (eagg,) = _seg(etab, tailp, headp, typep, HALF_E, ACC_E,
                           rel=rel_emb)
        (uagg,) = _seg(etab, ucp, urp, uvp, HALF_E, ACC_E)
        (ucf,) = _seg(itab, ucp, urp, uvp, HALF_E, ACC_E)
        (icf,) = _seg(ucf.reshape(2 * HALF_E, EMB), urp, ucp, uvp,
                      HALF_I, ACC_I)
        eagg = eagg.reshape(2 * HALF_E, EMB)
        uagg = uagg.reshape(2 * HALF_E, EMB)
        icf = icf.reshape(2 * HALF_I, EMB)
        etab, e_res = _t1_call(2 * HALF_E)(eagg, deg, e_res)
        u_res = _t2_call(2 * HALF_E)(uagg, u_res)
        i_res = _t2_call(2 * HALF_I)(icf, i_res)
        itab = icf

    neg_flat = neg_item.reshape(-1).astype(_i32)
    idx_en = jnp.concatenate([neg_flat, pos_item.astype(_i32)])
    g_e = _gather(e_res, idx_en)
    g_i = _gather(i_res, idx_en)
    g_u = _gather(u_res, user.astype(_i32))

    ne3 = g_e[: B * NEG].reshape(B, NEG, EMB)
    ni3 = g_i[: B * NEG].reshape(B, NEG, EMB)
    pe = g_e[B * NEG:]
    pi = g_i[B * NEG:]
    l1, sq = _loss_call()(g_u, pe, pi, ne3, ni3)
    loss1 = l1[0, 0] / B
    reg = DECAY * sq[0, 0] / (2.0 * B)

    t_real = tri_head.shape[0]
    tp = -(-t_real // 2048) * 2048
    padh = jnp.zeros((tp - t_real,), _i32)
    idx_ht = jnp.concatenate([tri_head.astype(_i32), padh,
                              tri_tail.astype(_i32), padh])
    g_ht = _gather(entity_emb, idx_ht)
    asum = _angle_call(tp, t_real)(g_ht, g_ht)
    loss2 = ANGLE_W * asum[0, 0] / t_real

    return loss1 + reg + loss2


# speedup vs baseline: 2.1322x; 2.1322x over previous
import functools

import jax
import jax.numpy as jnp
from jax import lax
from jax.experimental import pallas as pl
from jax.experimental.pallas import tpu as pltpu
from jax.experimental.pallas import tpu_sc as plsc

N_USERS = 50000
N_ITEMS = 20000
N_ENTITIES = 50000
N_REL = 17
EMB = 64
HOPS = 2
B = 4096
NEG = 16
MARGIN = 0.8
DECAY = 1e-4
ANGLE_W = 0.5
ANGLE_DROP = 0.5

NCORES = 2
NTILES = 16
EP = 802816
SUPER = 1024
CHUNK = 128
NSUPER = EP // (NTILES * SUPER)

HALF_E = 25088
HALF_I = 10112
ACC_E = HALF_E + 64
ACC_I = HALF_I + 64
GARB = 63

_f32 = jnp.float32
_i32 = jnp.int32


@functools.lru_cache(maxsize=None)
def _seg_op(n_half, acc_rows, mode, with_deg):
    mesh = plsc.VectorSubcoreMesh(core_axis_name="c", subcore_axis_name="s")
    out_type = [jax.ShapeDtypeStruct((NCORES, n_half, EMB), _f32)]
    if with_deg:
        out_type.append(jax.ShapeDtypeStruct((NCORES, n_half, 8), _f32))
    scratch = [
        pltpu.VMEM((SUPER,), _i32),
        pltpu.VMEM((SUPER,), _f32 if mode == "scalar" else _i32),
        pltpu.VMEM((CHUNK,), _i32),
        pltpu.VMEM((CHUNK, EMB), _f32),
        pltpu.VMEM_SHARED((acc_rows, EMB), _f32),
        pltpu.SemaphoreType.DMA,
    ]
    if mode == "rel":
        scratch.append(pltpu.VMEM((N_REL - 1, EMB), _f32))
    if with_deg:
        scratch.append(pltpu.VMEM((CHUNK, 8), _f32))
        scratch.append(pltpu.VMEM_SHARED((acc_rows, 8), _f32))

    def body(*refs):
        it = iter(refs)
        table = next(it); src = next(it); dst = next(it); w = next(it)
        rel = next(it) if mode == "rel" else None
        ones_h = next(it) if with_deg else None
        z64 = next(it); z8 = next(it)
        out = next(it)
        dout = next(it) if with_deg else None
        sbuf = next(it); wbuf = next(it); dbuf = next(it); rows = next(it)
        acc = next(it); sem = next(it)
        relbuf = next(it) if mode == "rel" else None
        onesbuf = next(it) if with_deg else None
        dacc = next(it) if with_deg else None

        c = lax.axis_index("c")
        s = lax.axis_index("s")

        zfull = acc_rows // 2048
        ztail = acc_rows - zfull * 2048
        for j in range(-(-zfull // NTILES)):
            cid = j * NTILES + s

            @pl.when(cid < zfull)
            def _():
                base = cid * 2048
                pltpu.sync_copy(z64, acc.at[pl.ds(base, 2048)])
                if with_deg:
                    pltpu.sync_copy(z8, dacc.at[pl.ds(base, 2048)])
        if ztail:
            @pl.when(s == NTILES - 1)
            def _():
                base = zfull * 2048
                pltpu.sync_copy(z64.at[pl.ds(0, ztail)],
                                acc.at[pl.ds(base, ztail)])
                if with_deg:
                    pltpu.sync_copy(z8.at[pl.ds(0, ztail)],
                                    dacc.at[pl.ds(base, ztail)])
        if mode == "rel":
            pltpu.sync_copy(rel, relbuf)
        if with_deg:
            pltpu.sync_copy(ones_h, onesbuf)
        plsc.subcore_barrier()

        half_base = c * n_half

        def chunk_body(k, sj):
            base = sj + k * CHUNK
            pltpu.sync_copy(dst.at[pl.ds(base, CHUNK)], dbuf)
            pltpu.async_copy(
                table.at[sbuf.at[pl.ds(k * CHUNK, CHUNK)]], rows, sem).wait()
            for g in range(CHUNK // 16):
                dv = dbuf[pl.ds(g * 16, 16)]
                loc = dv - half_base
                ok = (loc >= 0) & (loc < n_half)
                garb = n_half + (dv & GARB)
                dbuf[pl.ds(g * 16, 16)] = jnp.where(ok, loc, garb)
            for g in range(CHUNK // 16):
                w16 = wbuf[pl.ds(k * CHUNK + g * 16, 16)]
                for e in range(16):
                    m = g * 16 + e
                    if mode == "scalar":
                        wm = w16[e]
                        for q in range(EMB // 16):
                            sl = rows[m, pl.ds(q * 16, 16)]
                            rows[m, pl.ds(q * 16, 16)] = sl * wm
                    else:
                        tm = w16[e] - 1
                        for q in range(EMB // 16):
                            sl = rows[m, pl.ds(q * 16, 16)]
                            rv = relbuf[tm, pl.ds(q * 16, 16)]
                            rows[m, pl.ds(q * 16, 16)] = sl * rv
            pltpu.sync_copy(rows, acc.at[dbuf], add=True)
            if with_deg:
                pltpu.sync_copy(onesbuf, dacc.at[dbuf], add=True)
            return sj

        def super_body(j, carry):
            sbase = (s * NSUPER + j) * SUPER
            pltpu.sync_copy(src.at[pl.ds(sbase, SUPER)], sbuf)
            pltpu.sync_copy(w.at[pl.ds(sbase, SUPER)], wbuf)
            lax.fori_loop(0, SUPER // CHUNK, chunk_body, sbase,
                          unroll=False)
            return carry

        lax.fori_loop(0, NSUPER, super_body, 0, unroll=False)
        plsc.subcore_barrier()

        nchunks = n_half // CHUNK
        for j in range(-(-nchunks // NTILES)):
            cid = j * NTILES + s

            @pl.when(cid < nchunks)
            def _():
                r0 = cid * CHUNK
                pltpu.sync_copy(acc.at[pl.ds(r0, CHUNK)],
                                out.at[c, pl.ds(r0, CHUNK)])
                if with_deg:
                    pltpu.sync_copy(dacc.at[pl.ds(r0, CHUNK)],
                                    dout.at[c, pl.ds(r0, CHUNK)])

    return pl.kernel(body, out_type=out_type, mesh=mesh,
                     scratch_types=scratch,
                     compiler_params=pltpu.CompilerParams(
                         use_tc_tiling_on_sc=False))


def _seg(table, src, dst, w, n_half, acc_rows, rel=None, with_deg=False):
    z64 = jnp.zeros((2048, EMB), _f32)
    z8 = jnp.zeros((2048, 8), _f32)
    mode = "rel" if rel is not None else "scalar"
    k = _seg_op(n_half, acc_rows, mode, with_deg)
    args = [table, src, dst, w]
    if rel is not None:
        args.append(rel)
    if with_deg:
        ones = jnp.zeros((CHUNK, 8), _f32).at[:, 0].set(1.0)
        args.append(ones)
    args += [z64, z8]
    out = k(*args)
    return out if isinstance(out, (tuple, list)) else (out,)


@functools.lru_cache(maxsize=None)
def _gather_op(n_rows, n_idx):
    mesh = plsc.VectorSubcoreMesh(core_axis_name="c", subcore_axis_name="s")
    nchunks = n_idx // (NCORES * NTILES * CHUNK)

    def body(table, idx, out, ibuf, rows, sem):
        c = lax.axis_index("c")
        s = lax.axis_index("s")
        wid = s * NCORES + c

        def chunk(j, carry):
            base = (wid * nchunks + j) * CHUNK
            pltpu.sync_copy(idx.at[pl.ds(base, CHUNK)], ibuf)
            pltpu.async_copy(table.at[ibuf], rows, sem).wait()
            pltpu.sync_copy(rows, out.at[pl.ds(base, CHUNK)])
            return carry

        lax.fori_loop(0, nchunks, chunk, 0, unroll=False)

    return pl.kernel(
        body,
        out_type=jax.ShapeDtypeStruct((n_idx, EMB), _f32),
        mesh=mesh,
        scratch_types=[pltpu.VMEM((CHUNK,), _i32),
                       pltpu.VMEM((CHUNK, EMB), _f32),
                       pltpu.SemaphoreType.DMA],
        compiler_params=pltpu.CompilerParams(use_tc_tiling_on_sc=False))


def _gather(table, idx):
    return _gather_op(table.shape[0], idx.shape[0])(table, idx)


def _t1_body(agg_ref, deg_ref, res_ref, e1_ref, out_ref):
    d = jnp.maximum(deg_ref[:, 0:1], 1.0)
    x = agg_ref[...] / d
    e1_ref[...] = x
    nrm = jnp.sqrt(jnp.sum(x * x, axis=1, keepdims=True)) + 1e-8
    out_ref[...] = res_ref[...] + x / nrm


def _t2_body(x_ref, res_ref, out_ref):
    x = x_ref[...]
    nrm = jnp.sqrt(jnp.sum(x * x, axis=1, keepdims=True)) + 1e-8
    out_ref[...] = res_ref[...] + x / nrm


@functools.lru_cache(maxsize=None)
def _t1_call(n_rows):
    blk = 512
    grid = n_rows // blk
    return pl.pallas_call(
        _t1_body,
        grid=(grid,),
        in_specs=[pl.BlockSpec((blk, EMB), lambda i: (i, 0)),
                  pl.BlockSpec((blk, 8), lambda i: (i, 0)),
                  pl.BlockSpec((blk, EMB), lambda i: (i, 0))],
        out_specs=[pl.BlockSpec((blk, EMB), lambda i: (i, 0)),
                   pl.BlockSpec((blk, EMB), lambda i: (i, 0))],
        out_shape=[jax.ShapeDtypeStruct((n_rows, EMB), _f32),
                   jax.ShapeDtypeStruct((n_rows, EMB), _f32)])


@functools.lru_cache(maxsize=None)
def _t2_call(n_rows):
    blk = 512 if n_rows % 512 == 0 else 256
    grid = n_rows // blk
    return pl.pallas_call(
        _t2_body,
        grid=(grid,),
        in_specs=[pl.BlockSpec((blk, EMB), lambda i: (i, 0)),
                  pl.BlockSpec((blk, EMB), lambda i: (i, 0))],
        out_specs=pl.BlockSpec((blk, EMB), lambda i: (i, 0)),
        out_shape=jax.ShapeDtypeStruct((n_rows, EMB), _f32))


def _loss_body(u_ref, pe_ref, pi_ref, ne_ref, ni_ref, l1_ref, sq_ref):
    i = pl.program_id(0)
    u_e = u_ref[...]
    pe = pe_ref[...]
    pi = pi_ref[...]
    ne = ne_ref[...]
    ni = ni_ref[...]
    u = u_e / (jnp.sqrt(jnp.sum(u_e * u_e, -1, keepdims=True)) + 1e-8)
    ps = pe + pi
    p = ps / (jnp.sqrt(jnp.sum(ps * ps, -1, keepdims=True)) + 1e-8)
    pos_score = jnp.sum(u * p, -1)
    ns = ne + ni
    n = ns / (jnp.sqrt(jnp.sum(ns * ns, -1, keepdims=True)) + 1e-8)
    neg_score = jnp.sum(u[:, None, :] * n, -1)
    hinge = (jnp.maximum(1.0 - pos_score, 0.0)
             + jnp.mean(jnp.maximum(neg_score - MARGIN, 0.0), -1))
    part1 = jnp.sum(hinge)
    sq = (jnp.sum(u_e * u_e) + jnp.sum(pe * pe) + jnp.sum(pi * pi)
          + jnp.sum(ne * ne) + jnp.sum(ni * ni))

    @pl.when(i == 0)
    def _():
        l1_ref[0, 0] = 0.0
        sq_ref[0, 0] = 0.0

    l1_ref[0, 0] += part1
    sq_ref[0, 0] += sq


def _loss_call():
    blk = 512
    grid = B // blk
    return pl.pallas_call(
        _loss_body,
        grid=(grid,),
        in_specs=[pl.BlockSpec((blk, EMB), lambda i: (i, 0)),
                  pl.BlockSpec((blk, EMB), lambda i: (i, 0)),
                  pl.BlockSpec((blk, EMB), lambda i: (i, 0)),
                  pl.BlockSpec((blk, NEG, EMB), lambda i: (i, 0, 0)),
                  pl.BlockSpec((blk, NEG, EMB), lambda i: (i, 0, 0))],
        out_specs=[pl.BlockSpec(memory_space=pltpu.SMEM),
                   pl.BlockSpec(memory_space=pltpu.SMEM)],
        out_shape=[jax.ShapeDtypeStruct((1, 1), _f32),
                   jax.ShapeDtypeStruct((1, 1), _f32)])


def _acos_poly(x):
    a = jnp.abs(x)
    s = jnp.sqrt(1.0 - a)
    p = jnp.float32(-0.0012624911)
    for cc in (0.0066700901, -0.0170881256, 0.0308918810, -0.0501743046,
               0.0889789874, -0.2145988016, 1.5707963050):
        p = p * a + jnp.float32(cc)
    b = s * p
    return jnp.where(x >= 0, b, jnp.float32(jnp.pi) - b)


def _angle_body(t_real, h_ref, t_ref, out_ref):
    i = pl.program_id(0)
    blk = h_ref.shape[0]
    h = h_ref[...] * ANGLE_DROP
    t = t_ref[...] * ANGLE_DROP
    eps = 1e-6
    k_const = 0.1
    hh = jnp.sum(h * h, -1)
    tt = jnp.sum(t * t, -1)
    dot = jnp.sum(h * t, -1)
    d = h - t
    edist = jnp.sqrt(jnp.sum(d * d, -1))
    nu = jnp.sqrt(hh)
    num = dot * (1.0 + hh) - hh * (1.0 + tt)
    den = nu * edist * jnp.sqrt(jnp.clip(1.0 + tt * hh - 2.0 * dot, eps)) + eps
    ang = _acos_poly(jnp.clip(num / den, -1.0 + eps, 1.0 - eps))
    sqnu = jnp.clip(hh, 0.0, 1.0 - eps)
    asin_arg = jnp.clip(k_const * (1.0 - sqnu) / jnp.sqrt(sqnu + eps),
                        -1.0 + eps, 1.0 - eps)
    half = jnp.float32(jnp.pi / 2) - _acos_poly(asin_arg)
    val = jnp.maximum(ang - half, 0.0)
    rid = i * blk + lax.broadcasted_iota(_i32, (blk,), 0)
    part = jnp.sum(jnp.where(rid < t_real, val, 0.0))

    @pl.when(i == 0)
    def _():
        out_ref[0, 0] = 0.0

    out_ref[0, 0] += part


@functools.lru_cache(maxsize=None)
def _angle_call(tp, t_real):
    blk = 2048
    grid = tp // blk
    return pl.pallas_call(
        functools.partial(_angle_body, t_real),
        grid=(grid,),
        in_specs=[pl.BlockSpec((blk, EMB), lambda i: (i, 0)),
                  pl.BlockSpec((blk, EMB), lambda i, g=grid: (g + i, 0))],
        out_specs=pl.BlockSpec(memory_space=pltpu.SMEM),
        out_shape=jax.ShapeDtypeStruct((1, 1), _f32))


def _pad_rows(x, n):
    return jnp.zeros((n, EMB), _f32).at[: x.shape[0]].set(x)


def _pad_edges(x, val, dtype):
    return jnp.concatenate(
        [x.astype(dtype), jnp.full((EP - x.shape[0],), val, dtype)])


def kernel(user, pos_item, neg_item, all_embed, item_emb_cf, rel_emb,
           edge_index, edge_type, ui_rows, ui_cols, ui_vals,
           tri_head, tri_tail):
    user_emb = all_embed[:N_USERS]
    entity_emb = all_embed[N_USERS:]

    headp = _pad_edges(edge_index[0], -1, _i32)
    tailp = _pad_edges(edge_index[1], 0, _i32)
    typep = _pad_edges(edge_type, 1, _i32)
    ur_src = _pad_edges(ui_rows, 0, _i32)
    ur_dst = _pad_edges(ui_rows, -1, _i32)
    uc_src = _pad_edges(ui_cols, 0, _i32)
    uc_dst = _pad_edges(ui_cols, -1, _i32)
    uvp = _pad_edges(ui_vals, 0.0, _f32)

    e_res = _pad_rows(entity_emb, 2 * HALF_E)
    u_res = _pad_rows(user_emb, 2 * HALF_E)
    i_res = _pad_rows(item_emb_cf, 2 * HALF_I)

    etab = entity_emb
    itab = item_emb_cf
    deg = None
    for hop in range(HOPS):
        if hop == 0:
            eagg, deg = _seg(etab, tailp, headp, typep, HALF_E, ACC_E,
                             rel=rel_emb, with_deg=True)
            deg = deg.reshape(2 * HALF_E, 8)
        else:
            (eagg,) = _seg(etab, tailp, headp, typep, HALF_E, ACC_E,
                           rel=rel_emb)
        (uagg,) = _seg(etab, uc_src, ur_dst, uvp, HALF_E, ACC_E)
        (ucf,) = _seg(itab, uc_src, ur_dst, uvp, HALF_E, ACC_E)
        (icf,) = _seg(ucf.reshape(2 * HALF_E, EMB), ur_src, uc_dst, uvp,
                      HALF_I, ACC_I)
        eagg = eagg.reshape(2 * HALF_E, EMB)
        uagg = uagg.reshape(2 * HALF_E, EMB)
        icf = icf.reshape(2 * HALF_I, EMB)
        etab, e_res = _t1_call(2 * HALF_E)(eagg, deg, e_res)
        u_res = _t2_call(2 * HALF_E)(uagg, u_res)
        i_res = _t2_call(2 * HALF_I)(icf, i_res)
        itab = icf

    neg_flat = neg_item.reshape(-1).astype(_i32)
    idx_en = jnp.concatenate([neg_flat, pos_item.astype(_i32)])
    g_e = _gather(e_res, idx_en)
    g_i = _gather(i_res, idx_en)
    g_u = _gather(u_res, user.astype(_i32))

    ne3 = g_e[: B * NEG].reshape(B, NEG, EMB)
    ni3 = g_i[: B * NEG].reshape(B, NEG, EMB)
    pe = g_e[B * NEG:]
    pi = g_i[B * NEG:]
    l1, sq = _loss_call()(g_u, pe, pi, ne3, ni3)
    loss1 = l1[0, 0] / B
    reg = DECAY * sq[0, 0] / (2.0 * B)

    t_real = tri_head.shape[0]
    tp = -(-t_real // 2048) * 2048
    padh = jnp.zeros((tp - t_real,), _i32)
    idx_ht = jnp.concatenate([tri_head.astype(_i32), padh,
                              tri_tail.astype(_i32), padh])
    g_ht = _gather(entity_emb, idx_ht)
    asum = _angle_call(tp, t_real)(g_ht, g_ht)
    loss2 = ANGLE_W * asum[0, 0] / t_real

    return loss1 + reg + loss2
